# baseline (speedup 1.0000x reference)
import jax
import jax.numpy as jnp
from jax.experimental import pallas as pl
from jax.experimental.pallas import tpu as pltpu

DIM = 768
ROWS_PER_BLOCK = 2048


def _body(q_ref, m_ref, ax_ref, st_ref, o_ref, ms_ref, at_ref, b_ref):
    @pl.when(pl.program_id(0) == 0)
    def _prep():
        m = m_ref[...]
        ri = jax.lax.broadcasted_iota(jnp.int32, (DIM, DIM), 0)
        ci = jax.lax.broadcasted_iota(jnp.int32, (DIM, DIM), 1)
        ms_ref[...] = (0.4 * m).astype(jnp.bfloat16)
        del ri, ci
        ax = ax_ref[...]
        an = ax * jax.lax.rsqrt(
            jnp.maximum(jnp.sum(ax * ax, axis=1, keepdims=True), 1e-24))
        at_ref[...] = an.T.astype(jnp.bfloat16)
        s = jnp.maximum(st_ref[...], 0.0) + 1e-06
        w = s / jnp.sum(s)
        b_ref[...] = ((0.6 * w).T * an).astype(jnp.bfloat16)

    q = q_ref[...]
    qb = q.astype(jnp.bfloat16)
    p = jnp.dot(qb, ms_ref[...], preferred_element_type=jnp.float32)
    t = jnp.dot(qb, at_ref[...], preferred_element_type=jnp.float32)
    rinv = jax.lax.rsqrt(jnp.maximum(
        jnp.sum((qb * qb).astype(jnp.float32), axis=1, keepdims=True),
        1e-24))
    c = (t * rinv).astype(jnp.bfloat16)
    y = p + jnp.dot(c, b_ref[...], preferred_element_type=jnp.float32)
    yn = jax.lax.rsqrt(
        jnp.maximum(jnp.sum(y * y, axis=1, keepdims=True), 1e-24))
    o_ref[...] = y * yn


def kernel(Q, axes, strength, metric):
    B, S, D = Q.shape
    n = B * S
    K = axes.shape[0]
    Q2 = Q.reshape(n, D)
    st2 = strength.reshape(1, K)

    grid = (n // ROWS_PER_BLOCK,)
    out = pl.pallas_call(
        _body,
        grid=grid,
        in_specs=[
            pl.BlockSpec((ROWS_PER_BLOCK, D), lambda i: (i, 0)),
            pl.BlockSpec((D, D), lambda i: (0, 0)),
            pl.BlockSpec((K, D), lambda i: (0, 0)),
            pl.BlockSpec((1, K), lambda i: (0, 0)),
        ],
        out_specs=pl.BlockSpec((ROWS_PER_BLOCK, D), lambda i: (i, 0)),
        out_shape=jax.ShapeDtypeStruct((n, D), jnp.float32),
        scratch_shapes=[
            pltpu.VMEM((D, D), jnp.bfloat16),
            pltpu.VMEM((D, K), jnp.bfloat16),
            pltpu.VMEM((K, D), jnp.bfloat16),
        ],
        compiler_params=pltpu.CompilerParams(
            dimension_semantics=("arbitrary",),
        ),
    )(Q2, metric, axes, st2)
    return out.reshape(B, S, D)


# speedup vs baseline: 1.1804x; 1.1804x over previous
import jax
import jax.numpy as jnp
from jax.experimental import pallas as pl
from jax.experimental.pallas import tpu as pltpu

DIM = 768
ROWS_PER_BLOCK = 2048


def _body(q_ref, m_ref, ax_ref, st_ref, o_ref, ms_ref, at_ref, b_ref):
    @pl.when(pl.program_id(0) == 0)
    def _prep():
        m = m_ref[...]
        ri = jax.lax.broadcasted_iota(jnp.int32, (DIM, DIM), 0)
        ci = jax.lax.broadcasted_iota(jnp.int32, (DIM, DIM), 1)
        ms_ref[...] = (0.4 * m - jnp.where(ri == ci, 0.4, 0.0)).astype(
            jnp.bfloat16)
        ax = ax_ref[...]
        an = ax * jax.lax.rsqrt(
            jnp.maximum(jnp.sum(ax * ax, axis=1, keepdims=True), 1e-24))
        at_ref[...] = an.T.astype(jnp.bfloat16)
        s = jnp.maximum(st_ref[...], 0.0) + 1e-06
        w = s / jnp.sum(s)
        b_ref[...] = ((0.6 * w).T * an).astype(jnp.bfloat16)

    q = q_ref[...]
    qb = q.astype(jnp.bfloat16)
    p = 0.4 * q + jnp.dot(qb, ms_ref[...], preferred_element_type=jnp.float32)
    t = jnp.dot(qb, at_ref[...], preferred_element_type=jnp.float32)
    rinv = jax.lax.rsqrt(jnp.maximum(
        jnp.sum((qb * qb).astype(jnp.float32), axis=1, keepdims=True),
        1e-24))
    c = (t * rinv).astype(jnp.bfloat16)
    y = p + jnp.dot(c, b_ref[...], preferred_element_type=jnp.float32)
    yn = jax.lax.rsqrt(
        jnp.maximum(jnp.sum(y * y, axis=1, keepdims=True), 1e-24))
    o_ref[...] = y * yn


def kernel(Q, axes, strength, metric):
    B, S, D = Q.shape
    n = B * S
    K = axes.shape[0]
    Q2 = Q.reshape(n, D)
    st2 = strength.reshape(1, K)

    grid = (n // ROWS_PER_BLOCK,)
    out = pl.pallas_call(
        _body,
        grid=grid,
        in_specs=[
            pl.BlockSpec((ROWS_PER_BLOCK, D), lambda i: (i, 0)),
            pl.BlockSpec((D, D), lambda i: (0, 0)),
            pl.BlockSpec((K, D), lambda i: (0, 0)),
            pl.BlockSpec((1, K), lambda i: (0, 0)),
        ],
        out_specs=pl.BlockSpec((ROWS_PER_BLOCK, D), lambda i: (i, 0)),
        out_shape=jax.ShapeDtypeStruct((n, D), jnp.float32),
        scratch_shapes=[
            pltpu.VMEM((D, D), jnp.bfloat16),
            pltpu.VMEM((D, K), jnp.bfloat16),
            pltpu.VMEM((K, D), jnp.bfloat16),
        ],
        compiler_params=pltpu.CompilerParams(
            dimension_semantics=("arbitrary",),
        ),
    )(Q2, metric, axes, st2)
    return out.reshape(B, S, D)
